# baseline (speedup 1.0000x reference)
import functools

import jax
import jax.numpy as jnp
from jax import lax
from jax.experimental import pallas as pl
from jax.experimental.pallas import tpu as pltpu
from jax.experimental.pallas import tpu_sc as plsc

_NC = 2
_NS = 16
_NW = _NC * _NS
_K = 2


def _make_seg_sum(n, e, d):
    epw = e // _NW
    b = 80
    nchunk = epw // b
    nsec = 5
    spc = nchunk // nsec
    rpt0 = (n // _NS + 7) // 8 * 8
    rows_of = lambda s: min(rpt0, n - s * rpt0)

    mesh = plsc.VectorSubcoreMesh(core_axis_name="c", subcore_axis_name="s")

    @functools.partial(
        pl.kernel,
        out_type=jax.ShapeDtypeStruct((_NC, n, d), jnp.float32),
        mesh=mesh,
        scratch_types=[
            pltpu.VMEM((spc, b), jnp.int32),
            pltpu.VMEM((spc, b), jnp.int32),
            [pltpu.VMEM((b, d), jnp.float32) for _ in range(_K)],
            [pltpu.SemaphoreType.DMA for _ in range(_K)],
            [pltpu.SemaphoreType.DMA for _ in range(_K)],
            pltpu.VMEM_SHARED((n, d), jnp.float32),
        ],
    )
    def seg_sum(x_hbm, src_hbm, dst_hbm, out_hbm,
                src_v, dst_v, rows, gsems, ssems, acc_sh):
        cid = lax.axis_index("c")
        sid = lax.axis_index("s")
        wid = cid * _NS + sid

        for s in range(_NS):
            @pl.when(sid == s)
            def _():
                pltpu.sync_copy(x_hbm.at[pl.ds(s * rpt0, rows_of(s))],
                                acc_sh.at[pl.ds(s * rpt0, rows_of(s))])
        plsc.subcore_barrier()

        def fire_gather(c, j):
            pltpu.async_copy(x_hbm.at[src_v.at[c]], rows[j], gsems[j])

        def wait_gather(j):
            pltpu.make_async_copy(x_hbm.at[pl.ds(0, b)], rows[j],
                                  gsems[j]).wait()

        def fire_scat(c, j):
            pltpu.async_copy(rows[j], acc_sh.at[dst_v.at[c]], ssems[j],
                             add=True)

        def wait_scat(j):
            pltpu.make_async_copy(rows[j], acc_sh.at[dst_v.at[0]],
                                  ssems[j]).wait()

        ngrp = spc // _K
        tail = spc - ngrp * _K

        def section(sec, carry):
            pltpu.sync_copy(src_hbm.at[wid, sec], src_v)
            pltpu.sync_copy(dst_hbm.at[wid, sec], dst_v)
            for j in range(_K):
                fire_gather(j, j)

            def step(t, carry2):
                c0 = t * _K
                for j in range(_K):
                    wait_gather(j)
                    fire_scat(c0 + j, j)
                for j in range(_K):
                    wait_scat(j)
                    fire_gather(c0 + _K + j, j)
                return carry2

            lax.fori_loop(0, ngrp - 1, step, 0)
            c0 = (ngrp - 1) * _K
            for j in range(_K):
                wait_gather(j)
                fire_scat(c0 + j, j)
            for j in range(_K):
                wait_scat(j)
                if j < tail:
                    fire_gather(c0 + _K + j, j)
            for j in range(tail):
                wait_gather(j)
                fire_scat(c0 + _K + j, j)
            for j in range(tail):
                wait_scat(j)
            return carry

        lax.fori_loop(0, nsec, section, 0)
        plsc.subcore_barrier()

        for s in range(_NS):
            @pl.when(sid == s)
            def _():
                pltpu.sync_copy(acc_sh.at[pl.ds(s * rpt0, rows_of(s))],
                                out_hbm.at[cid, pl.ds(s * rpt0, rows_of(s))])

    return seg_sum


def _mlp_body(relu, x_ref, p_ref, w1_ref, b1_ref, g_ref, bb_ref,
              w2_ref, b2_ref, o_ref):
    a = p_ref[0] + p_ref[1] - x_ref[...]
    z = jnp.dot(a, w1_ref[...], preferred_element_type=jnp.float32) + b1_ref[...]
    mu = jnp.mean(z, axis=0, keepdims=True)
    var = jnp.mean((z - mu) ** 2, axis=0, keepdims=True)
    z = (z - mu) / jnp.sqrt(var + 1e-5) * g_ref[...] + bb_ref[...]
    if relu:
        z = jnp.maximum(z, 0.0)
    o_ref[...] = jnp.dot(z, w2_ref[...], preferred_element_type=jnp.float32) + b2_ref[...]


def _mlp(x, p, params, relu):
    n, d = x.shape
    h = params["W2"].shape[1]
    return pl.pallas_call(
        functools.partial(_mlp_body, relu),
        out_shape=jax.ShapeDtypeStruct((n, h), jnp.float32),
    )(x, p, params["W1"], params["b1"], params["g"], params["b"],
      params["W2"], params["b2"])


def _final_body(h_ref, p_ref, noise_ref,
                mw1, mb1, mg, mbb, mw2, mb2,
                lw1, lb1, lg, lbb, lw2, lb2, o_ref):
    a = p_ref[0] + p_ref[1] - h_ref[...]

    def mlp(w1, b1, g, bb, w2, b2):
        z = jnp.dot(a, w1[...], preferred_element_type=jnp.float32) + b1[...]
        mu = jnp.mean(z, axis=0, keepdims=True)
        var = jnp.mean((z - mu) ** 2, axis=0, keepdims=True)
        z = (z - mu) / jnp.sqrt(var + 1e-5) * g[...] + bb[...]
        return jnp.dot(z, w2[...], preferred_element_type=jnp.float32) + b2[...]

    mean = mlp(mw1, mb1, mg, mbb, mw2, mb2)
    logstd = mlp(lw1, lb1, lg, lbb, lw2, lb2)
    o_ref[...] = noise_ref[...] * jnp.exp(logstd) + mean


def kernel(x, edge_index, gaussian_noise, params):
    n, d = x.shape
    e = edge_index.shape[1]
    b = 80
    nchunk = e // _NW // b
    nsec = 5
    src = edge_index[0].reshape(_NW, nsec, nchunk // nsec, b)
    dst = edge_index[1].reshape(_NW, nsec, nchunk // nsec, b)

    seg_sum = _make_seg_sum(n, e, d)

    p1 = seg_sum(x, src, dst)
    h1 = _mlp(x, p1, params["l1"], True)
    p2 = seg_sum(h1, src, dst)
    h2 = _mlp(h1, p2, params["l2"], True)
    p3 = seg_sum(h2, src, dst)

    pm = params["mu"]
    pls = params["ls"]
    return pl.pallas_call(
        _final_body,
        out_shape=jax.ShapeDtypeStruct((n, d), jnp.float32),
    )(h2, p3, gaussian_noise,
      pm["W1"], pm["b1"], pm["g"], pm["b"], pm["W2"], pm["b2"],
      pls["W1"], pls["b1"], pls["g"], pls["b"], pls["W2"], pls["b2"])


# speedup vs baseline: 1.2083x; 1.2083x over previous
import functools

import jax
import jax.numpy as jnp
from jax import lax
from jax.experimental import pallas as pl
from jax.experimental.pallas import tpu as pltpu
from jax.experimental.pallas import tpu_sc as plsc

_NC = 2
_NS = 16
_NW = _NC * _NS
_K = 3


def _make_seg_sum(n, e, d):
    epw = e // _NW
    b = 80
    nchunk = epw // b
    nsec = 5
    spc = nchunk // nsec
    rpt0 = (n // _NS + 7) // 8 * 8
    rows_of = lambda s: min(rpt0, n - s * rpt0)

    mesh = plsc.VectorSubcoreMesh(core_axis_name="c", subcore_axis_name="s")

    @functools.partial(
        pl.kernel,
        out_type=jax.ShapeDtypeStruct((_NC, n, d), jnp.float32),
        mesh=mesh,
        scratch_types=[
            pltpu.VMEM((spc, b), jnp.int32),
            pltpu.VMEM((spc, b), jnp.int32),
            [pltpu.VMEM((b, d), jnp.float32) for _ in range(_K)],
            [pltpu.SemaphoreType.DMA for _ in range(_K)],
            [pltpu.SemaphoreType.DMA for _ in range(_K)],
            pltpu.SemaphoreType.DMA,
            pltpu.VMEM_SHARED((n, d), jnp.float32),
        ],
    )
    def seg_sum(x_hbm, src_hbm, dst_hbm, out_hbm,
                src_v, dst_v, rows, gsems, ssems, isem, acc_sh):
        cid = lax.axis_index("c")
        sid = lax.axis_index("s")
        wid = cid * _NS + sid

        pltpu.async_copy(src_hbm.at[wid, 0], src_v, isem)
        pltpu.async_copy(dst_hbm.at[wid, 0], dst_v, isem)
        for s in range(_NS):
            @pl.when(sid == s)
            def _():
                pltpu.sync_copy(x_hbm.at[pl.ds(s * rpt0, rows_of(s))],
                                acc_sh.at[pl.ds(s * rpt0, rows_of(s))])
        plsc.subcore_barrier()

        def fire_gather(c, j):
            pltpu.async_copy(x_hbm.at[src_v.at[c]], rows[j], gsems[j])

        def wait_gather(j):
            pltpu.make_async_copy(x_hbm.at[pl.ds(0, b)], rows[j],
                                  gsems[j]).wait()

        def fire_scat(c, j):
            pltpu.async_copy(rows[j], acc_sh.at[dst_v.at[c]], ssems[j],
                             add=True)

        def wait_scat(j):
            pltpu.make_async_copy(rows[j], acc_sh.at[dst_v.at[0]],
                                  ssems[j]).wait()

        ngrp = spc // _K
        tail = spc - ngrp * _K

        def section(sec, carry):
            pltpu.make_async_copy(src_hbm.at[wid, 0], src_v, isem).wait()
            pltpu.make_async_copy(dst_hbm.at[wid, 0], dst_v, isem).wait()
            for j in range(_K):
                fire_gather(j, j)

            def step(t, carry2):
                c0 = t * _K
                for j in range(_K):
                    wait_gather(j)
                    fire_scat(c0 + j, j)
                for j in range(_K):
                    wait_scat(j)
                    fire_gather(c0 + _K + j, j)
                return carry2

            lax.fori_loop(0, ngrp - 1, step, 0)
            c0 = (ngrp - 1) * _K
            for j in range(_K):
                wait_gather(j)
                fire_scat(c0 + j, j)
            for j in range(_K):
                wait_scat(j)
                if j < tail:
                    fire_gather(c0 + _K + j, j)
            for j in range(tail):
                wait_gather(j)
                fire_scat(c0 + _K + j, j)

            @pl.when(sec + 1 < nsec)
            def _():
                pltpu.async_copy(src_hbm.at[wid, sec + 1], src_v, isem)
            for j in range(tail):
                wait_scat(j)
            @pl.when(sec + 1 < nsec)
            def _():
                pltpu.async_copy(dst_hbm.at[wid, sec + 1], dst_v, isem)
            return carry

        lax.fori_loop(0, nsec, section, 0)
        plsc.subcore_barrier()

        for s in range(_NS):
            @pl.when(sid == s)
            def _():
                pltpu.sync_copy(acc_sh.at[pl.ds(s * rpt0, rows_of(s))],
                                out_hbm.at[cid, pl.ds(s * rpt0, rows_of(s))])

    return seg_sum


def _mlp_body(relu, x_ref, p_ref, w1_ref, b1_ref, g_ref, bb_ref,
              w2_ref, b2_ref, o_ref):
    a = p_ref[0] + p_ref[1] - x_ref[...]
    z = jnp.dot(a, w1_ref[...], preferred_element_type=jnp.float32) + b1_ref[...]
    mu = jnp.mean(z, axis=0, keepdims=True)
    var = jnp.mean((z - mu) ** 2, axis=0, keepdims=True)
    z = (z - mu) / jnp.sqrt(var + 1e-5) * g_ref[...] + bb_ref[...]
    if relu:
        z = jnp.maximum(z, 0.0)
    o_ref[...] = jnp.dot(z, w2_ref[...], preferred_element_type=jnp.float32) + b2_ref[...]


def _mlp(x, p, params, relu):
    n, d = x.shape
    h = params["W2"].shape[1]
    return pl.pallas_call(
        functools.partial(_mlp_body, relu),
        out_shape=jax.ShapeDtypeStruct((n, h), jnp.float32),
    )(x, p, params["W1"], params["b1"], params["g"], params["b"],
      params["W2"], params["b2"])


def _final_body(h_ref, p_ref, noise_ref,
                mw1, mb1, mg, mbb, mw2, mb2,
                lw1, lb1, lg, lbb, lw2, lb2, o_ref):
    a = p_ref[0] + p_ref[1] - h_ref[...]

    def mlp(w1, b1, g, bb, w2, b2):
        z = jnp.dot(a, w1[...], preferred_element_type=jnp.float32) + b1[...]
        mu = jnp.mean(z, axis=0, keepdims=True)
        var = jnp.mean((z - mu) ** 2, axis=0, keepdims=True)
        z = (z - mu) / jnp.sqrt(var + 1e-5) * g[...] + bb[...]
        return jnp.dot(z, w2[...], preferred_element_type=jnp.float32) + b2[...]

    mean = mlp(mw1, mb1, mg, mbb, mw2, mb2)
    logstd = mlp(lw1, lb1, lg, lbb, lw2, lb2)
    o_ref[...] = noise_ref[...] * jnp.exp(logstd) + mean


def kernel(x, edge_index, gaussian_noise, params):
    n, d = x.shape
    e = edge_index.shape[1]
    b = 80
    nchunk = e // _NW // b
    nsec = 5
    src = edge_index[0].reshape(_NW, nsec, nchunk // nsec, b)
    dst = edge_index[1].reshape(_NW, nsec, nchunk // nsec, b)

    seg_sum = _make_seg_sum(n, e, d)

    p1 = seg_sum(x, src, dst)
    h1 = _mlp(x, p1, params["l1"], True)
    p2 = seg_sum(h1, src, dst)
    h2 = _mlp(h1, p2, params["l2"], True)
    p3 = seg_sum(h2, src, dst)

    pm = params["mu"]
    pls = params["ls"]
    return pl.pallas_call(
        _final_body,
        out_shape=jax.ShapeDtypeStruct((n, d), jnp.float32),
    )(h2, p3, gaussian_noise,
      pm["W1"], pm["b1"], pm["g"], pm["b"], pm["W2"], pm["b2"],
      pls["W1"], pls["b1"], pls["g"], pls["b"], pls["W2"], pls["b2"])
